# baseline (speedup 1.0000x reference)
import functools

import jax
import jax.numpy as jnp
from jax import lax
from jax.experimental import pallas as pl
from jax.experimental.pallas import tpu as pltpu
from jax.experimental.pallas import tpu_sc as plsc

BATCH = 16384
ARITY = 2
EMBED_DIM = 64
HIDDEN = 256

ROWS = BATCH * ARITY
NC, NS = 2, 16
NW = NC * NS
ROWS_PER_W = ROWS // NW
CHUNK = 128
NCHUNK = ROWS_PER_W // CHUNK


def _sc_gather(table, idx3):
    mesh = plsc.VectorSubcoreMesh(core_axis_name="c", subcore_axis_name="s")

    @functools.partial(
        pl.kernel,
        out_type=jax.ShapeDtypeStruct((ROWS, EMBED_DIM), jnp.float32),
        mesh=mesh,
        scratch_types=[
            pltpu.VMEM((NCHUNK, CHUNK), jnp.int32),
            pltpu.VMEM((ROWS_PER_W, EMBED_DIM), jnp.float32),
            pltpu.SemaphoreType.DMA,
        ],
        compiler_params=pltpu.CompilerParams(use_tc_tiling_on_sc=False),
    )
    def gather_kernel(table_hbm, idx_hbm, out_hbm, idx_v, rows_v, sem):
        wid = lax.axis_index("s") * NC + lax.axis_index("c")
        base = wid * ROWS_PER_W
        pltpu.sync_copy(idx_hbm.at[wid], idx_v)
        copies = [
            pltpu.async_copy(
                table_hbm.at[idx_v.at[j]],
                rows_v.at[pl.ds(j * CHUNK, CHUNK)],
                sem,
            )
            for j in range(NCHUNK)
        ]
        for c in copies:
            c.wait()
        pltpu.sync_copy(rows_v, out_hbm.at[pl.ds(base, ROWS_PER_W)])

    return gather_kernel(table, idx3)


def _mlp_body(x_ref, w1_ref, b1_ref, w2t_ref, b2_ref, o_ref):
    x = x_ref[...]
    h = jnp.dot(x, w1_ref[...], preferred_element_type=jnp.float32)
    h = jnp.maximum(h + b1_ref[...][None, :], 0.0)
    logit = jnp.sum(h * w2t_ref[...], axis=1) + b2_ref[0]
    o_ref[...] = jax.nn.sigmoid(logit)


def _tc_mlp(x, W1, b1, W2t, b2):
    TB = 1024
    grid = (BATCH // TB,)
    return pl.pallas_call(
        _mlp_body,
        grid=grid,
        in_specs=[
            pl.BlockSpec((TB, ARITY * EMBED_DIM), lambda i: (i, 0)),
            pl.BlockSpec((ARITY * EMBED_DIM, HIDDEN), lambda i: (0, 0)),
            pl.BlockSpec((HIDDEN,), lambda i: (0,)),
            pl.BlockSpec((1, HIDDEN), lambda i: (0, 0)),
            pl.BlockSpec(memory_space=pltpu.SMEM),
        ],
        out_specs=pl.BlockSpec((TB,), lambda i: (i,)),
        out_shape=jax.ShapeDtypeStruct((BATCH,), jnp.float32),
    )(x, W1, b1, W2t, b2)


def kernel(indices, table, W1, b1, W2, b2):
    idx3 = indices.astype(jnp.int32).reshape(NW, NCHUNK, CHUNK)
    gathered = _sc_gather(table, idx3)
    x = gathered.reshape(BATCH, ARITY * EMBED_DIM)
    return _tc_mlp(x, W1, b1, W2.reshape(1, HIDDEN), b2)


# speedup vs baseline: 2.3065x; 2.3065x over previous
import functools

import jax
import jax.numpy as jnp
from jax import lax
from jax.experimental import pallas as pl
from jax.experimental.pallas import tpu as pltpu
from jax.experimental.pallas import tpu_sc as plsc

BATCH = 16384
ARITY = 2
EMBED_DIM = 64
HIDDEN = 256
NUM_CONST = 1000000

ROWS = BATCH * ARITY
NC, NS = 2, 16
NW = NC * NS
ROWS_PER_W = ROWS // NW
GROUP = 8
FIRE = 32
NBATCH = ROWS_PER_W // FIRE


def _sc_gather(tbl3, idx2):
    mesh = plsc.VectorSubcoreMesh(core_axis_name="c", subcore_axis_name="s")

    @functools.partial(
        pl.kernel,
        out_type=jax.ShapeDtypeStruct((BATCH, ARITY * EMBED_DIM), jnp.float32),
        mesh=mesh,
        scratch_types=[
            pltpu.VMEM((ROWS_PER_W,), jnp.int32),
            pltpu.VMEM((ROWS_PER_W // 2, ARITY * EMBED_DIM), jnp.float32),
            pltpu.SemaphoreType.DMA,
        ],
        compiler_params=pltpu.CompilerParams(needs_layout_passes=False),
    )
    def gather_kernel(tbl_hbm, idx_hbm, out_hbm, idx_v, rows_v, sem):
        wid = lax.axis_index("s") * NC + lax.axis_index("c")
        pltpu.sync_copy(idx_hbm.at[wid], idx_v)
        out_base = wid * (ROWS_PER_W // 2)

        def batch_body(b, carry):
            copies = []
            for k2 in range(FIRE // 16):
                iv = idx_v[pl.ds(b * FIRE + k2 * 16, 16)]
                for k1 in range(16):
                    k = k2 * 16 + k1
                    v = iv[k1]
                    copies.append(
                        pltpu.async_copy(
                            tbl_hbm.at[jnp.right_shift(v, 3),
                                       jnp.bitwise_and(v, 7)],
                            rows_v.at[b * (FIRE // 2) + k // 2,
                                      pl.ds((k % 2) * EMBED_DIM, EMBED_DIM)],
                            sem,
                        )
                    )
            for c in copies:
                c.wait()
            return carry

        lax.fori_loop(0, NBATCH, batch_body, 0)
        pltpu.sync_copy(
            rows_v, out_hbm.at[pl.ds(out_base, ROWS_PER_W // 2)]
        )

    return gather_kernel(tbl3, idx2)


def _mlp_body(x_ref, w1_ref, b1_ref, w2t_ref, b2_ref, o_ref):
    xb = x_ref[...].astype(jnp.bfloat16)
    w1b = w1_ref[...].astype(jnp.bfloat16)
    h = jnp.dot(xb, w1b, preferred_element_type=jnp.float32)
    h = jnp.maximum(h + b1_ref[...][None, :], 0.0)
    logit = jnp.sum(h * w2t_ref[...], axis=1) + b2_ref[0]
    o_ref[...] = jax.nn.sigmoid(logit)


def _tc_mlp(x, W1, b1, W2t, b2):
    TB = 1024
    grid = (BATCH // TB,)
    return pl.pallas_call(
        _mlp_body,
        grid=grid,
        in_specs=[
            pl.BlockSpec((TB, ARITY * EMBED_DIM), lambda i: (i, 0)),
            pl.BlockSpec((ARITY * EMBED_DIM, HIDDEN), lambda i: (0, 0)),
            pl.BlockSpec((HIDDEN,), lambda i: (0,)),
            pl.BlockSpec((1, HIDDEN), lambda i: (0, 0)),
            pl.BlockSpec(memory_space=pltpu.SMEM),
        ],
        out_specs=pl.BlockSpec((TB,), lambda i: (i,)),
        out_shape=jax.ShapeDtypeStruct((BATCH,), jnp.float32),
    )(x, W1, b1, W2t, b2)


def kernel(indices, table, W1, b1, W2, b2):
    idx2 = indices.astype(jnp.int32).reshape(NW, ROWS_PER_W)
    tbl3 = table.reshape(NUM_CONST // GROUP, GROUP, EMBED_DIM)
    x = _sc_gather(tbl3, idx2)
    return _tc_mlp(x, W1, b1, W2.reshape(1, HIDDEN), b2)
